# baseline (speedup 1.0000x reference)
import jax
import jax.numpy as jnp
from jax.experimental import pallas as pl
from jax.experimental.pallas import tpu as pltpu

_NUM_BITS = 16
_NUM_CAT = 1 << _NUM_BITS
_C_BLK = 4096


def _onehot_body(logits_ref, u_ref, out_ref):
    j = pl.program_id(0)
    logits = logits_ref[...]
    u = u_ref[...]
    bits = (u < jax.nn.sigmoid(logits)).astype(jnp.int32)
    pow2 = jnp.left_shift(
        1, jax.lax.broadcasted_iota(jnp.int32, logits.shape, 1)
    )
    idx = jnp.sum(bits * pow2, axis=1)
    cols = jax.lax.broadcasted_iota(
        jnp.int32, (logits.shape[0], _C_BLK), 1
    ) + j * _C_BLK
    out_ref[...] = (idx[:, None] == cols).astype(jnp.float32)


def kernel(bit_logits):
    b, s, h = bit_logits.shape
    t = b * s
    u = jax.random.uniform(
        jax.random.key(42), bit_logits.shape, dtype=bit_logits.dtype
    )
    out = pl.pallas_call(
        _onehot_body,
        grid=(_NUM_CAT // _C_BLK,),
        in_specs=[
            pl.BlockSpec((t, h), lambda j: (0, 0)),
            pl.BlockSpec((t, h), lambda j: (0, 0)),
        ],
        out_specs=pl.BlockSpec((t, _C_BLK), lambda j: (0, j)),
        out_shape=jax.ShapeDtypeStruct((t, _NUM_CAT), jnp.float32),
    )(bit_logits.reshape(t, h), u.reshape(t, h))
    return out.reshape(b, s, _NUM_CAT)
